# Initial kernel scaffold; baseline (speedup 1.0000x reference)
import jax
import jax.numpy as jnp
from jax.experimental import pallas as pl


def kernel(q, k, v, R):
    raise NotImplementedError("write your pallas kernel here")


# speedup vs baseline: 5.8899x; 5.8899x over previous
import functools
import math

import jax
import jax.numpy as jnp
from jax import lax
from jax.experimental import pallas as pl
from jax.experimental.pallas import tpu as pltpu
from jax.experimental.pallas import tpu_sc as plsc

_NUM_HASH = 16
_BLOCK = 256
_SAMPLE = 256
_IDXW = 128
_HASH_ROWS = 2048
_LANES = 128


def _hash_body(q_ref, k_ref, rp_ref, w_ref, qh_ref, kh_ref):
    rp = rp_ref[...]
    w = w_ref[...]
    for x_ref, o_ref in ((q_ref, qh_ref), (k_ref, kh_ref)):
        proj = jnp.dot(x_ref[...], rp, preferred_element_type=jnp.float32)
        code = jnp.sum(jnp.where(proj > 0, w, 0.0), axis=-1, keepdims=True)
        o_ref[...] = code.astype(jnp.int32)


def _hash_codes(q2, k2, R):
    n, d = q2.shape
    rp = jnp.zeros((d, _LANES), jnp.float32).at[:, :_NUM_HASH].set(R)
    w = (2.0 ** jnp.arange(_LANES, dtype=jnp.float32))[None, :]
    w = jnp.where(jnp.arange(_LANES)[None, :] < _NUM_HASH, w, 0.0)
    grid = (n // _HASH_ROWS,)
    qh, kh = pl.pallas_call(
        _hash_body,
        grid=grid,
        in_specs=[
            pl.BlockSpec((_HASH_ROWS, d), lambda i: (i, 0)),
            pl.BlockSpec((_HASH_ROWS, d), lambda i: (i, 0)),
            pl.BlockSpec((d, _LANES), lambda i: (0, 0)),
            pl.BlockSpec((1, _LANES), lambda i: (0, 0)),
        ],
        out_specs=[
            pl.BlockSpec((_HASH_ROWS, 1), lambda i: (i, 0)),
            pl.BlockSpec((_HASH_ROWS, 1), lambda i: (i, 0)),
        ],
        out_shape=[
            jax.ShapeDtypeStruct((n, 1), jnp.int32),
            jax.ShapeDtypeStruct((n, 1), jnp.int32),
        ],
    )(q2, k2, rp, w)
    return qh[:, 0], kh[:, 0]


def _sc_gather(q2, k2, v2, qidx, kidx):
    n, d = q2.shape
    info = plsc.get_sparse_core_info()
    nw = info.num_cores * info.num_subcores
    nch = n // _IDXW // nw
    mesh = plsc.VectorSubcoreMesh(core_axis_name="c", subcore_axis_name="s")

    @functools.partial(
        pl.kernel,
        out_type=(jax.ShapeDtypeStruct((n, d), jnp.float32),) * 3,
        mesh=mesh,
        scratch_types=[
            pltpu.VMEM((nch, _IDXW), jnp.int32),
            pltpu.VMEM((nch, _IDXW), jnp.int32),
            pltpu.VMEM((_IDXW, d), jnp.float32),
            pltpu.VMEM((_IDXW, d), jnp.float32),
            pltpu.VMEM((_IDXW, d), jnp.float32),
            pltpu.SemaphoreType.DMA,
            pltpu.SemaphoreType.DMA,
            pltpu.SemaphoreType.DMA,
        ],
        compiler_params=pltpu.CompilerParams(use_tc_tiling_on_sc=False),
    )
    def gather(qf, kf, vf, qi_hbm, ki_hbm, qs, ks, vs,
               qi_v, ki_v, qrow, krow, vrow, sq, sk, sv):
        wid = lax.axis_index("s") * info.num_cores + lax.axis_index("c")
        rbase = wid * nch
        pltpu.sync_copy(qi_hbm.at[pl.ds(rbase, nch)], qi_v)
        pltpu.sync_copy(ki_hbm.at[pl.ds(rbase, nch)], ki_v)

        def body(c, carry):
            base = (rbase + c) * _IDXW
            cq = pltpu.async_copy(qf.at[qi_v.at[c]], qrow, sq)
            ck = pltpu.async_copy(kf.at[ki_v.at[c]], krow, sk)
            cv = pltpu.async_copy(vf.at[ki_v.at[c]], vrow, sv)
            cq.wait()
            pltpu.sync_copy(qrow, qs.at[pl.ds(base, _IDXW)])
            ck.wait()
            pltpu.sync_copy(krow, ks.at[pl.ds(base, _IDXW)])
            cv.wait()
            pltpu.sync_copy(vrow, vs.at[pl.ds(base, _IDXW)])
            return carry

        lax.fori_loop(0, nch, body, 0)

    return gather(q2, k2, v2, qidx, kidx)


def _attn_body(scale, log_ratio, qs_ref, ks_ref, vs_ref, ksm_ref, vsm_ref, o_ref):
    qb = qs_ref[0]
    kb = ks_ref[0]
    vb = vs_ref[0]
    ksm = ksm_ref[0]
    vsm = vsm_ref[0]

    dn = (((1,), (1,)), ((), ()))
    s1 = lax.dot_general(qb, kb, dn, preferred_element_type=jnp.float32) * scale
    m1 = jnp.max(s1, axis=-1, keepdims=True)
    p1 = jnp.exp(s1 - m1)
    d1 = jnp.sum(p1, axis=-1, keepdims=True)
    o1 = jnp.dot(p1, vb, preferred_element_type=jnp.float32)
    lse1 = m1 + jnp.log(d1)

    s2 = lax.dot_general(qb, ksm, dn, preferred_element_type=jnp.float32) * scale
    m2 = jnp.max(s2, axis=-1, keepdims=True)
    p2 = jnp.exp(s2 - m2)
    d2 = jnp.sum(p2, axis=-1, keepdims=True)
    o2 = jnp.dot(p2, vsm, preferred_element_type=jnp.float32)
    lse2 = m2 + jnp.log(d2) + log_ratio

    mx = jnp.maximum(lse1, lse2)
    w1 = jnp.exp(lse1 - mx)
    w2 = jnp.exp(lse2 - mx)
    o_ref[0] = (w1 * (o1 / d1) + w2 * (o2 / d2)) / (w1 + w2)


def _attn(qs, ks, vs, ksm, vsm):
    bh, s, d = qs.shape
    scale = 1.0 / math.sqrt(d)
    log_ratio = math.log(s / _SAMPLE)
    grid = (bh, s // _BLOCK)
    return pl.pallas_call(
        functools.partial(_attn_body, scale, log_ratio),
        grid=grid,
        in_specs=[
            pl.BlockSpec((1, _BLOCK, d), lambda h, b: (h, b, 0)),
            pl.BlockSpec((1, _BLOCK, d), lambda h, b: (h, b, 0)),
            pl.BlockSpec((1, _BLOCK, d), lambda h, b: (h, b, 0)),
            pl.BlockSpec((1, _SAMPLE, d), lambda h, b: (h, 0, 0)),
            pl.BlockSpec((1, _SAMPLE, d), lambda h, b: (h, 0, 0)),
        ],
        out_specs=pl.BlockSpec((1, _BLOCK, d), lambda h, b: (h, b, 0)),
        out_shape=jax.ShapeDtypeStruct((bh, s, d), jnp.float32),
    )(qs, ks, vs, ksm, vsm)


def _sc_scatter(o2, qidx):
    n, d = o2.shape
    info = plsc.get_sparse_core_info()
    nw = info.num_cores * info.num_subcores
    nch = n // _IDXW // nw
    mesh = plsc.VectorSubcoreMesh(core_axis_name="c", subcore_axis_name="s")

    @functools.partial(
        pl.kernel,
        out_type=jax.ShapeDtypeStruct((n, d), jnp.float32),
        mesh=mesh,
        scratch_types=[
            pltpu.VMEM((nch, _IDXW), jnp.int32),
            pltpu.VMEM((_IDXW, d), jnp.float32),
            pltpu.SemaphoreType.DMA,
        ],
        compiler_params=pltpu.CompilerParams(use_tc_tiling_on_sc=False),
    )
    def scatter(of, qi_hbm, out, qi_v, row, sem):
        wid = lax.axis_index("s") * info.num_cores + lax.axis_index("c")
        rbase = wid * nch
        pltpu.sync_copy(qi_hbm.at[pl.ds(rbase, nch)], qi_v)

        def body(c, carry):
            base = (rbase + c) * _IDXW
            pltpu.sync_copy(of.at[pl.ds(base, _IDXW)], row)
            pltpu.async_copy(row, out.at[qi_v.at[c]], sem).wait()
            return carry

        lax.fori_loop(0, nch, body, 0)

    return scatter(o2, qidx)


def kernel(q, k, v, R):
    b, h, s, d = q.shape
    bh = b * h
    q2 = q.reshape(bh * s, d)
    k2 = k.reshape(bh * s, d)
    v2 = v.reshape(bh * s, d)

    qh, kh = _hash_codes(q2, k2, R)
    qh = qh.reshape(bh, s)
    kh = kh.reshape(bh, s)

    q_perm = jnp.argsort(qh, axis=-1)
    k_perm = jnp.argsort(kh, axis=-1)
    offs = (jnp.arange(bh, dtype=jnp.int32) * s)[:, None]
    qidx = (q_perm.astype(jnp.int32) + offs).reshape(-1, _IDXW)
    kidx = (k_perm.astype(jnp.int32) + offs).reshape(-1, _IDXW)

    qs, ks, vs = _sc_gather(q2, k2, v2, qidx, kidx)

    stride = s // _SAMPLE
    ksm = k.reshape(bh, s, d)[:, ::stride, :]
    vsm = v.reshape(bh, s, d)[:, ::stride, :]

    o_sorted = _attn(qs.reshape(bh, s, d), ks.reshape(bh, s, d),
                     vs.reshape(bh, s, d), ksm, vsm)

    out = _sc_scatter(o_sorted.reshape(bh * s, d), qidx)
    return out.reshape(b, h, s, d)
